# baseline (speedup 1.0000x reference)
import functools

import jax
import jax.numpy as jnp
from jax.experimental import pallas as pl
from jax.experimental.pallas import tpu as pltpu


def _nc_kernel(params_ref, xa_ref, xb_ref, o_ref, *, inv_n, inv_nm1, nh):
    mean_t = params_ref[0]
    std_t = params_ref[1]
    min_v = params_ref[2]
    max_v = params_ref[3]

    xa = xa_ref[...].astype(jnp.float32)
    xb = xb_ref[...].astype(jnp.float32)
    s = (jnp.sum(xa, axis=-1, keepdims=True)
         + jnp.sum(xb, axis=-1, keepdims=True))
    sq = (jnp.sum(xa * xa, axis=-1, keepdims=True)
          + jnp.sum(xb * xb, axis=-1, keepdims=True))
    mu = s * inv_n
    var = (sq - s * mu) * inv_nm1
    gain = std_t * jax.lax.rsqrt(var)
    shift = gain * (mean_t - mu)
    ya = xa * gain + shift
    yb = xb * gain + shift
    o_ref[:, :nh] = jnp.minimum(jnp.maximum(ya, min_v), max_v).astype(o_ref.dtype)
    o_ref[:, nh:] = jnp.minimum(jnp.maximum(yb, min_v), max_v).astype(o_ref.dtype)


@jax.jit
def _normalize_clamp(x, mean, std, min_val, max_val):
    B, C, H, W = x.shape
    N = C * H * W
    nh = N // 2
    x2d = x.reshape(B, N)

    params = jnp.stack([
        jnp.asarray(mean, jnp.float32), jnp.asarray(std, jnp.float32),
        jnp.asarray(min_val, jnp.float32), jnp.asarray(max_val, jnp.float32)])

    tb = 16 if B % 16 == 0 else (8 if B > 8 else B)
    out2d = pl.pallas_call(
        functools.partial(_nc_kernel, inv_n=1.0 / N, inv_nm1=1.0 / (N - 1),
                          nh=nh),
        out_shape=jax.ShapeDtypeStruct((B, N), x.dtype),
        grid=(pl.cdiv(B, tb),),
        in_specs=[pl.BlockSpec(memory_space=pltpu.MemorySpace.SMEM),
                  pl.BlockSpec((tb, nh), lambda b: (b, 0)),
                  pl.BlockSpec((tb, nh), lambda b: (b, 1))],
        out_specs=pl.BlockSpec((tb, N), lambda b: (b, 0)),
        compiler_params=pltpu.CompilerParams(
            dimension_semantics=("arbitrary",),
            vmem_limit_bytes=56 * 1024 * 1024),
    )(params, x2d, x2d)
    return out2d.reshape(B, C, H, W)


def kernel(x, mean, std, min_val, max_val):
    return _normalize_clamp(x, mean, std, min_val, max_val)
